# Initial kernel scaffold; baseline (speedup 1.0000x reference)
import jax
import jax.numpy as jnp
from jax.experimental import pallas as pl


def kernel(x, edge_index, batch, W1, b1, W2, b2, Wc1, bc1, Wc2, bc2):
    raise NotImplementedError("write your pallas kernel here")


# speedup vs baseline: 13.5509x; 13.5509x over previous
import functools

import jax
import jax.numpy as jnp
from jax import lax
from jax.experimental import pallas as pl
from jax.experimental.pallas import tpu as pltpu
from jax.experimental.pallas import tpu_sc as plsc

N = 10000
E = 320000
D = 128
B = 64

NC = 2
NS = 16
EPT = E // (NC * NS)
CH = 80
NCH = EPT // CH
RPT = 624
ZR = 24
RB = 104

_mesh = plsc.VectorSubcoreMesh(core_axis_name="c", subcore_axis_name="s")


_DEG_ZB = 208


@functools.partial(
    pl.kernel,
    out_type=jax.ShapeDtypeStruct((NC * N,), jnp.float32),
    mesh=_mesh,
    scratch_types=[
        pltpu.MemorySpace.VMEM_SHARED((N,), jnp.float32),
        pltpu.MemorySpace.VMEM((CH,), jnp.int32),
        pltpu.MemorySpace.VMEM((CH,), jnp.float32),
        pltpu.MemorySpace.VMEM((_DEG_ZB,), jnp.float32),
        pltpu.MemorySpace.VMEM((624,), jnp.float32),
    ],
)
def _deg_kernel(dst_hbm, out_hbm, acc, didx, ones_v, zb, rb):
    c = lax.axis_index("c")
    s = lax.axis_index("s")
    for q in range(_DEG_ZB // 16):
        zb[pl.ds(q * 16, 16)] = jnp.zeros((16,), jnp.float32)
    for q in range(CH // 16):
        ones_v[pl.ds(q * 16, 16)] = jnp.ones((16,), jnp.float32)
    for t in range(624 // _DEG_ZB):
        pltpu.sync_copy(zb, acc.at[pl.ds(s * 624 + t * _DEG_ZB, _DEG_ZB)])

    @pl.when(s == NS - 1)
    def _():
        pltpu.sync_copy(zb.at[pl.ds(0, 16)], acc.at[pl.ds(16 * 624, 16)])

    plsc.subcore_barrier()

    ebase = (c * NS + s) * EPT

    def body(i, carry):
        base = ebase + i * CH
        pltpu.sync_copy(dst_hbm.at[pl.ds(base, CH)], didx)
        pltpu.sync_copy(ones_v, acc.at[didx], add=True)
        return carry

    lax.fori_loop(0, NCH, body, 0)
    plsc.subcore_barrier()
    pltpu.sync_copy(acc.at[pl.ds(s * 624, 624)], rb)
    pltpu.sync_copy(rb, out_hbm.at[pl.ds(c * N + s * 624, 624)])

    @pl.when(s == NS - 1)
    def _():
        pltpu.sync_copy(acc.at[pl.ds(16 * 624, 16)], rb.at[pl.ds(0, 16)])
        pltpu.sync_copy(rb.at[pl.ds(0, 16)],
                        out_hbm.at[pl.ds(c * N + 16 * 624, 16)])


@functools.partial(
    pl.kernel,
    out_type=jax.ShapeDtypeStruct((NC * N, D), jnp.float32),
    mesh=_mesh,
    scratch_types=[
        pltpu.MemorySpace.VMEM_SHARED((N, D), jnp.float32),
        pltpu.MemorySpace.VMEM((CH,), jnp.int32),
        pltpu.MemorySpace.VMEM((CH,), jnp.int32),
        pltpu.MemorySpace.VMEM((CH, D), jnp.float32),
        pltpu.MemorySpace.VMEM((ZR, D), jnp.float32),
        pltpu.MemorySpace.VMEM((RB, D), jnp.float32),
        pltpu.SemaphoreType.DMA,
    ],
)
def _spmm_kernel(u_hbm, src_hbm, dst_hbm, out_hbm, acc, sidx, didx, rows, zrow, rb, sem):
    c = lax.axis_index("c")
    s = lax.axis_index("s")
    for r in range(ZR):
        for q in range(D // 16):
            zrow[r, pl.ds(q * 16, 16)] = jnp.zeros((16,), jnp.float32)
    for t in range(RPT // ZR):
        pltpu.sync_copy(zrow, acc.at[pl.ds(s * RPT + t * ZR, ZR)])

    @pl.when(s == NS - 1)
    def _():
        pltpu.sync_copy(zrow.at[pl.ds(0, 16)], acc.at[pl.ds(NS * RPT, 16)])

    plsc.subcore_barrier()

    ebase = (c * NS + s) * EPT

    def body(i, carry):
        base = ebase + i * CH
        pltpu.sync_copy(src_hbm.at[pl.ds(base, CH)], sidx)
        pltpu.sync_copy(dst_hbm.at[pl.ds(base, CH)], didx)
        pltpu.async_copy(u_hbm.at[sidx], rows, sem).wait()
        pltpu.sync_copy(rows, acc.at[didx], add=True)
        return carry

    lax.fori_loop(0, NCH, body, 0)
    plsc.subcore_barrier()
    for t in range(RPT // RB):
        pltpu.sync_copy(acc.at[pl.ds(s * RPT + t * RB, RB)], rb)
        pltpu.sync_copy(rb, out_hbm.at[pl.ds(c * N + s * RPT + t * RB, RB)])

    @pl.when(s == NS - 1)
    def _():
        pltpu.sync_copy(acc.at[pl.ds(NS * RPT, 16)], rb.at[pl.ds(0, 16)])
        pltpu.sync_copy(rb.at[pl.ds(0, 16)], out_hbm.at[pl.ds(c * N + NS * RPT, 16)])


def _tc1_body(x_ref, w1t_ref, degp_ref, u1_ref, dinvb_ref):
    deg = degp_ref[0, :] + degp_ref[1, :] + 1.0
    dinv = lax.rsqrt(deg)
    dinvb = jnp.broadcast_to(dinv[:, None], (N, D))
    dinvb_ref[...] = dinvb
    z1 = jnp.dot(x_ref[...], w1t_ref[...], preferred_element_type=jnp.float32)
    u1_ref[...] = z1 * dinvb


def _tc2_body(sp_ref, u1_ref, dinvb_ref, b1_ref, w2t_ref, u2_ref):
    dinvb = dinvb_ref[...]
    h1 = jnp.maximum(dinvb * (sp_ref[0] + sp_ref[1] + u1_ref[...]) + b1_ref[...], 0.0)
    z2 = jnp.dot(h1, w2t_ref[...], preferred_element_type=jnp.float32)
    u2_ref[...] = z2 * dinvb


def _tc3_body(sp_ref, u2_ref, dinvb_ref, b2_ref, batch_ref, wc1t_ref, bc1_ref,
              wc2t_ref, bc2_ref, v_ref):
    dinvb = dinvb_ref[...]
    h2 = jnp.maximum(dinvb * (sp_ref[0] + sp_ref[1] + u2_ref[...]) + b2_ref[...], 0.0)
    oh = (lax.broadcasted_iota(jnp.int32, (B, N), 0) == batch_ref[...]).astype(jnp.float32)
    sums = jnp.dot(oh, h2, preferred_element_type=jnp.float32)
    cnt = jnp.sum(oh, axis=1, keepdims=True)
    pooled = sums / jnp.maximum(cnt, 1.0)
    t = jnp.tanh(jnp.dot(pooled, wc1t_ref[...], preferred_element_type=jnp.float32)
                 + bc1_ref[...])
    v_ref[...] = jnp.dot(t, wc2t_ref[...], preferred_element_type=jnp.float32) + bc2_ref[...]


_tc1 = pl.pallas_call(
    _tc1_body,
    out_shape=(jax.ShapeDtypeStruct((N, D), jnp.float32),
               jax.ShapeDtypeStruct((N, D), jnp.float32)),
)

_tc2 = pl.pallas_call(
    _tc2_body,
    out_shape=jax.ShapeDtypeStruct((N, D), jnp.float32),
)

_tc3 = pl.pallas_call(
    _tc3_body,
    out_shape=jax.ShapeDtypeStruct((B, 1), jnp.float32),
)


def kernel(x, edge_index, batch, W1, b1, W2, b2, Wc1, bc1, Wc2, bc2):
    src = edge_index[0]
    dst = edge_index[1]
    degp = _deg_kernel(dst).reshape(2, N)
    u1, dinvb = _tc1(x, W1.T, degp)
    s1 = _spmm_kernel(u1, src, dst).reshape(2, N, D)
    u2 = _tc2(s1, u1, dinvb, b1.reshape(1, D), W2.T)
    s2 = _spmm_kernel(u2, src, dst).reshape(2, N, D)
    v = _tc3(s2, u2, dinvb, b2.reshape(1, D), batch.reshape(1, N),
             Wc1.T, bc1.reshape(1, D), Wc2.T, bc2.reshape(1, 1))
    return v
